# baseline (speedup 1.0000x reference)
import functools

import jax
import jax.numpy as jnp
from jax import lax
from jax.experimental import pallas as pl
from jax.experimental.pallas import tpu as pltpu
from jax.experimental.pallas import tpu_sc as plsc

BATCH = 4
SEQ = 4096
D = 768
NW = 32
POS_PER_W = SEQ // NW
STEP = 32
STEPS_PER_B = POS_PER_W // STEP
T = BATCH * STEPS_PER_B
BLK = D // 32
NBUF = 3
AHEAD = 1
SLACK = NBUF - AHEAD
PEEL = 4
UNROLL = 8


def _emb_kernel(x_hbm, table_hbm, posq_hbm, out_hbm,
                idx_v, pos_v, rows_v, sem_idx, sem_pos, sem_g, sem_st):
    cid = lax.axis_index("c")
    sid = lax.axis_index("s")
    wid = sid * 2 + cid
    ps = wid * POS_PER_W

    idx_cp = pltpu.async_copy(
        x_hbm.at[pl.ds(wid * BATCH * POS_PER_W, BATCH * POS_PER_W)],
        idx_v, sem_idx)
    pos_cp = pltpu.async_copy(posq_hbm.at[pl.ds(ps, POS_PER_W)], pos_v,
                              sem_pos)
    idx_cp.wait()

    def gather(t, buf):
        return pltpu.async_copy(
            table_hbm.at[idx_v.at[pl.ds(t * STEP, STEP)]],
            rows_v.at[buf], sem_g.at[buf])

    def out_row(t):
        return (t // STEPS_PER_B) * SEQ + ps + lax.rem(t, STEPS_PER_B) * STEP

    def drain_gather(b):
        pltpu.make_async_copy(
            table_hbm.at[idx_v.at[pl.ds(0, STEP)]],
            rows_v.at[b], sem_g.at[b]).wait()

    def drain_store(b):
        pltpu.make_async_copy(
            rows_v.at[b], out_hbm.at[pl.ds(0, STEP)], sem_st.at[b]).wait()

    def add_step(t, b):
        prow = lax.rem(t, STEPS_PER_B) * STEP

        def add_body(r, carry2):
            for k in range(BLK):
                w = pos_v[prow + r, pl.ds(k * 16, 16)]
                lo = lax.bitcast_convert_type(w << 16, jnp.float32)
                hi = lax.bitcast_convert_type(w & jnp.int32(-65536),
                                              jnp.float32)
                plsc.addupdate(rows_v.at[b, r, pl.ds(k * 32, 16)], lo)
                plsc.addupdate(rows_v.at[b, r, pl.ds(k * 32 + 16, 16)], hi)
            return carry2

        lax.fori_loop(0, STEP, add_body, 0, unroll=UNROLL)

    def do_step(t, tmod, first=False):
        pf_buf = (tmod + AHEAD) % NBUF
        if not first:
            drain_store(pf_buf)
        gather(lax.rem(t + AHEAD, T) if not isinstance(t, int)
               else (t + AHEAD) % T, pf_buf)
        drain_gather(tmod)
        add_step(t, tmod)
        pltpu.async_copy(
            rows_v.at[tmod], out_hbm.at[pl.ds(out_row(t), STEP)],
            sem_st.at[tmod])

    for t in range(AHEAD):
        gather(t, t % NBUF)
    pos_cp.wait()
    for t in range(PEEL):
        do_step(t, t % NBUF, first=t < SLACK)

    def step_tri(g, carry):
        for i in range(NBUF):
            t = PEEL + g * NBUF + i
            do_step(t, (PEEL + i) % NBUF)
        return carry

    lax.fori_loop(0, (T - PEEL) // NBUF, step_tri, 0)
    for t in range(T - SLACK, T):
        drain_store(t % NBUF)
    for i in range(AHEAD):
        drain_gather((T + i) % NBUF)


@jax.jit
def kernel(x, table, pos_encoding):
    flat_idx = (x.astype(jnp.int32)
                .reshape(BATCH, NW, POS_PER_W)
                .transpose(1, 0, 2)
                .reshape(-1))
    pbits = lax.bitcast_convert_type(
        pos_encoding.astype(jnp.bfloat16), jnp.uint16
    ).reshape(SEQ, BLK, 2, 16).astype(jnp.uint32)
    posq = (pbits[:, :, 1, :] << 16 | pbits[:, :, 0, :]).astype(
        jnp.int32).reshape(SEQ, BLK * 16)
    mesh = plsc.VectorSubcoreMesh(core_axis_name="c", subcore_axis_name="s")
    run = functools.partial(
        pl.kernel,
        out_type=jax.ShapeDtypeStruct((BATCH * SEQ, D), jnp.float32),
        mesh=mesh,
        scratch_types=[
            pltpu.VMEM((BATCH * POS_PER_W,), jnp.int32),
            pltpu.VMEM((POS_PER_W, BLK * 16), jnp.int32),
            pltpu.VMEM((NBUF, STEP, D), jnp.float32),
            pltpu.SemaphoreType.DMA,
            pltpu.SemaphoreType.DMA,
            pltpu.SemaphoreType.DMA((NBUF,)),
            pltpu.SemaphoreType.DMA((NBUF,)),
        ],
    )(_emb_kernel)
    out = run(flat_idx, table, posq)
    return out.reshape(BATCH, SEQ, D)


# speedup vs baseline: 1.0496x; 1.0496x over previous
import functools

import jax
import jax.numpy as jnp
from jax import lax
from jax.experimental import pallas as pl
from jax.experimental.pallas import tpu as pltpu
from jax.experimental.pallas import tpu_sc as plsc

BATCH = 4
SEQ = 4096
D = 768
NW = 32
POS_PER_W = SEQ // NW
STEP = 32
STEPS_PER_B = POS_PER_W // STEP
T = BATCH * STEPS_PER_B
BLK = D // 32
NBUF = 3
AHEAD = 1
SLACK = NBUF - AHEAD
PEEL = 4
UNROLL = 4


def _emb_kernel(x_hbm, table_hbm, posq_hbm, out_hbm,
                idx_v, pos_v, rows_v, sem_idx, sem_pos, sem_g, sem_st):
    cid = lax.axis_index("c")
    sid = lax.axis_index("s")
    wid = sid * 2 + cid
    ps = wid * POS_PER_W

    idx_cp = pltpu.async_copy(
        x_hbm.at[pl.ds(wid * BATCH * POS_PER_W, BATCH * POS_PER_W)],
        idx_v, sem_idx)
    pos_cp = pltpu.async_copy(posq_hbm.at[pl.ds(ps, POS_PER_W)], pos_v,
                              sem_pos)
    idx_cp.wait()

    def gather(t, buf):
        return pltpu.async_copy(
            table_hbm.at[idx_v.at[pl.ds(t * STEP, STEP)]],
            rows_v.at[buf], sem_g.at[buf])

    def out_row(t):
        return (t // STEPS_PER_B) * SEQ + ps + lax.rem(t, STEPS_PER_B) * STEP

    def drain_gather(b):
        pltpu.make_async_copy(
            table_hbm.at[idx_v.at[pl.ds(0, STEP)]],
            rows_v.at[b], sem_g.at[b]).wait()

    def drain_store(b):
        pltpu.make_async_copy(
            rows_v.at[b], out_hbm.at[pl.ds(0, STEP)], sem_st.at[b]).wait()

    def add_step(t, b):
        prow = lax.rem(t, STEPS_PER_B) * STEP

        def add_body(r, carry2):
            for k in range(BLK):
                w = pos_v[prow + r, pl.ds(k * 16, 16)]
                lo = lax.bitcast_convert_type(w << 16, jnp.float32)
                hi = lax.bitcast_convert_type(w, jnp.float32)
                plsc.addupdate(rows_v.at[b, r, pl.ds(k * 32, 16)], lo)
                plsc.addupdate(rows_v.at[b, r, pl.ds(k * 32 + 16, 16)], hi)
            return carry2

        lax.fori_loop(0, STEP, add_body, 0, unroll=UNROLL)

    def do_step(t, tmod, first=False):
        pf_buf = (tmod + AHEAD) % NBUF
        if not first:
            drain_store(pf_buf)
        gather(lax.rem(t + AHEAD, T) if not isinstance(t, int)
               else (t + AHEAD) % T, pf_buf)
        drain_gather(tmod)
        add_step(t, tmod)
        pltpu.async_copy(
            rows_v.at[tmod], out_hbm.at[pl.ds(out_row(t), STEP)],
            sem_st.at[tmod])

    for t in range(AHEAD):
        gather(t, t % NBUF)
    pos_cp.wait()
    for t in range(PEEL):
        do_step(t, t % NBUF, first=t < SLACK)

    def step_tri(g, carry):
        for i in range(NBUF):
            t = PEEL + g * NBUF + i
            do_step(t, (PEEL + i) % NBUF)
        return carry

    lax.fori_loop(0, (T - PEEL) // NBUF, step_tri, 0)
    for t in range(T - SLACK, T):
        drain_store(t % NBUF)
    for i in range(AHEAD):
        drain_gather((T + i) % NBUF)


@jax.jit
def kernel(x, table, pos_encoding):
    flat_idx = (x.astype(jnp.int32)
                .reshape(BATCH, NW, POS_PER_W)
                .transpose(1, 0, 2)
                .reshape(-1))
    pbits = lax.bitcast_convert_type(
        pos_encoding.astype(jnp.bfloat16), jnp.uint16
    ).reshape(SEQ, BLK, 2, 16).astype(jnp.uint32)
    posq = (pbits[:, :, 1, :] << 16 | pbits[:, :, 0, :]).astype(
        jnp.int32).reshape(SEQ, BLK * 16)
    mesh = plsc.VectorSubcoreMesh(core_axis_name="c", subcore_axis_name="s")
    run = functools.partial(
        pl.kernel,
        out_type=jax.ShapeDtypeStruct((BATCH * SEQ, D), jnp.float32),
        mesh=mesh,
        scratch_types=[
            pltpu.VMEM((BATCH * POS_PER_W,), jnp.int32),
            pltpu.VMEM((POS_PER_W, BLK * 16), jnp.int32),
            pltpu.VMEM((NBUF, STEP, D), jnp.float32),
            pltpu.SemaphoreType.DMA,
            pltpu.SemaphoreType.DMA,
            pltpu.SemaphoreType.DMA((NBUF,)),
            pltpu.SemaphoreType.DMA((NBUF,)),
        ],
    )(_emb_kernel)
    out = run(flat_idx, table, posq)
    return out.reshape(BATCH, SEQ, D)


# speedup vs baseline: 1.1626x; 1.1076x over previous
import functools

import jax
import jax.numpy as jnp
from jax import lax
from jax.experimental import pallas as pl
from jax.experimental.pallas import tpu as pltpu
from jax.experimental.pallas import tpu_sc as plsc

BATCH = 4
SEQ = 4096
D = 768
NW = 32
POS_PER_W = SEQ // NW
STEP = 32
POS_STEP = STEP // BATCH
T = POS_PER_W // POS_STEP
BLK = D // 32
NBUF = 3
AHEAD = 1
SLACK = NBUF - AHEAD
PEEL = 4
UNROLL = 4


def _emb_kernel(x_hbm, table_hbm, posq_hbm, out_hbm,
                idx_v, pos_v, rows_v, sem_idx, sem_pos, sem_g, sem_st):
    cid = lax.axis_index("c")
    sid = lax.axis_index("s")
    wid = sid * 2 + cid
    ps = wid * POS_PER_W

    idx_cp = pltpu.async_copy(
        x_hbm.at[pl.ds(wid * BATCH * POS_PER_W, BATCH * POS_PER_W)],
        idx_v, sem_idx)
    pos_cp = pltpu.async_copy(posq_hbm.at[pl.ds(ps, POS_PER_W)], pos_v,
                              sem_pos)
    idx_cp.wait()

    def gather(t, buf):
        return pltpu.async_copy(
            table_hbm.at[idx_v.at[pl.ds(t * STEP, STEP)]],
            rows_v.at[buf], sem_g.at[buf])


    def drain_gather(b):
        pltpu.make_async_copy(
            table_hbm.at[idx_v.at[pl.ds(0, STEP)]],
            rows_v.at[b], sem_g.at[b]).wait()

    def drain_store(b):
        pltpu.make_async_copy(
            rows_v.at[b], out_hbm.at[pl.ds(0, STEP)], sem_st.at[b]).wait()

    def add_step(t, b):
        prow = t * POS_STEP

        def add_body(r, carry2):
            for k in range(BLK):
                w = pos_v[prow + r, pl.ds(k * 16, 16)]
                lo = lax.bitcast_convert_type(w << 16, jnp.float32)
                hi = lax.bitcast_convert_type(w, jnp.float32)
                for bb in range(BATCH):
                    rr = bb * POS_STEP + r
                    plsc.addupdate(rows_v.at[b, rr, pl.ds(k * 32, 16)], lo)
                    plsc.addupdate(
                        rows_v.at[b, rr, pl.ds(k * 32 + 16, 16)], hi)
            return carry2

        lax.fori_loop(0, POS_STEP, add_body, 0, unroll=UNROLL)

    def do_step(t, tmod, first=False):
        pf_buf = (tmod + AHEAD) % NBUF
        if not first:
            drain_store(pf_buf)
        gather(lax.rem(t + AHEAD, T) if not isinstance(t, int)
               else (t + AHEAD) % T, pf_buf)
        drain_gather(tmod)
        add_step(t, tmod)
        for bb in range(BATCH):
            pltpu.async_copy(
                rows_v.at[tmod, pl.ds(bb * POS_STEP, POS_STEP)],
                out_hbm.at[pl.ds(bb * SEQ + ps + t * POS_STEP, POS_STEP)],
                sem_st.at[tmod])

    for t in range(AHEAD):
        gather(t, t % NBUF)
    pos_cp.wait()
    for t in range(PEEL):
        do_step(t, t % NBUF, first=t < SLACK)

    def step_tri(g, carry):
        for i in range(NBUF):
            t = PEEL + g * NBUF + i
            do_step(t, (PEEL + i) % NBUF)
        return carry

    lax.fori_loop(0, (T - PEEL) // NBUF, step_tri, 0)
    for t in range(T - SLACK, T):
        drain_store(t % NBUF)
    for i in range(AHEAD):
        drain_gather((T + i) % NBUF)


@jax.jit
def kernel(x, table, pos_encoding):
    flat_idx = (x.astype(jnp.int32)
                .reshape(BATCH, NW, POS_PER_W // (STEP // BATCH),
                         STEP // BATCH)
                .transpose(1, 2, 0, 3)
                .reshape(-1))
    pbits = lax.bitcast_convert_type(
        pos_encoding.astype(jnp.bfloat16), jnp.uint16
    ).reshape(SEQ, BLK, 2, 16).astype(jnp.uint32)
    posq = (pbits[:, :, 1, :] << 16 | pbits[:, :, 0, :]).astype(
        jnp.int32).reshape(SEQ, BLK * 16)
    mesh = plsc.VectorSubcoreMesh(core_axis_name="c", subcore_axis_name="s")
    run = functools.partial(
        pl.kernel,
        out_type=jax.ShapeDtypeStruct((BATCH * SEQ, D), jnp.float32),
        mesh=mesh,
        scratch_types=[
            pltpu.VMEM((BATCH * POS_PER_W,), jnp.int32),
            pltpu.VMEM((POS_PER_W, BLK * 16), jnp.int32),
            pltpu.VMEM((NBUF, STEP, D), jnp.float32),
            pltpu.SemaphoreType.DMA,
            pltpu.SemaphoreType.DMA,
            pltpu.SemaphoreType.DMA((NBUF,)),
            pltpu.SemaphoreType.DMA((NBUF,)),
        ],
    )(_emb_kernel)
    out = run(flat_idx, table, posq)
    return out.reshape(BATCH, SEQ, D)
